# Initial kernel scaffold; baseline (speedup 1.0000x reference)
import jax
import jax.numpy as jnp
from jax.experimental import pallas as pl


def kernel(input):
    raise NotImplementedError("write your pallas kernel here")


# speedup vs baseline: 9.0588x; 9.0588x over previous
import functools
import jax
import jax.numpy as jnp
from jax import lax
from jax.experimental import pallas as pl
from jax.experimental.pallas import tpu as pltpu

_BITS = 8
_RATIO = 0.125
_PARTITION = 4

_CAND = 0
_CNT = 1
_R = 2
_ICAND = 3
_C8 = 4
_N8 = 12


def _tqk_kernel(x_hbm, o_hbm, xv, ov, st, sem_in, sem_out, *,
                k, idx_bits, br, nb):
    i32min = jnp.int32(-2147483648)
    mant = jnp.int32(0x7FFFFFFF)
    one = jnp.int32(1)
    s = pl.program_id(0)
    b = pl.program_id(1)
    kk = jnp.int32(k)
    cs = k // _PARTITION
    targets = (1, cs, cs + 1, 2 * cs, 2 * cs + 1, 3 * cs, 3 * cs + 1, 4 * cs)
    n_t = 31
    s_gt = n_t
    s_tie0 = n_t + 1
    s_b0 = s_tie0 + idx_bits
    s_out = s_b0 + 32

    @pl.when(s == 0)
    def _stage_in():
        pltpu.make_async_copy(
            x_hbm.at[pl.ds(b * br, br), :],
            xv.at[pl.ds(b * br, br), :], sem_in).start()

    @pl.when((s == 0) & (b == 0))
    def _init():
        st[_CAND] = jnp.int32(0)
        st[_CNT] = jnp.int32(0)
        st[_ICAND] = jnp.int32(0)
        for j in range(8):
            st[_C8 + j] = jnp.int32(0)
            st[_N8 + j] = jnp.int32(0)

    @pl.when(s == 0)
    def _stage_wait():
        pltpu.make_async_copy(
            x_hbm.at[pl.ds(b * br, br), :],
            xv.at[pl.ds(b * br, br), :], sem_in).wait()

    C = xv.shape[1]

    def blk():
        return xv[pl.ds(b * br, br), :]

    def abs_key(xb):
        return lax.bitcast_convert_type(xb, jnp.int32) & mant

    def flat_idx():
        return (b * br * C
                + lax.broadcasted_iota(jnp.int32, (br, C), 0) * C
                + lax.broadcasted_iota(jnp.int32, (br, C), 1))

    @pl.when(s < n_t)
    def _thresh():
        trial = st[_CAND] | (one << (jnp.int32(30) - s))
        cnt = jnp.sum((abs_key(blk()) >= trial).astype(jnp.int32))
        new = st[_CNT] + cnt

        @pl.when(b != nb - 1)
        def _():
            st[_CNT] = new

        @pl.when(b == nb - 1)
        def _():
            st[_CAND] = jnp.where(new >= kk, trial, st[_CAND])
            st[_CNT] = jnp.int32(0)

    @pl.when(s == s_gt)
    def _cgt():
        cnt = jnp.sum((abs_key(blk()) > st[_CAND]).astype(jnp.int32))
        new = st[_CNT] + cnt

        @pl.when(b != nb - 1)
        def _():
            st[_CNT] = new

        @pl.when(b == nb - 1)
        def _():
            st[_R] = kk - new
            st[_CNT] = jnp.int32(0)

    @pl.when((s >= s_tie0) & (s < s_b0))
    def _tie():
        bit = jnp.int32(idx_bits - 1) - (s - jnp.int32(s_tie0))
        trial = st[_ICAND] | (one << bit)
        eq = abs_key(blk()) == st[_CAND]
        cnt = jnp.sum((eq & (flat_idx() < trial)).astype(jnp.int32))
        new = st[_CNT] + cnt

        @pl.when(b != nb - 1)
        def _():
            st[_CNT] = new

        @pl.when(b == nb - 1)
        def _():
            st[_ICAND] = jnp.where(new >= st[_R], st[_ICAND], trial)
            st[_CNT] = jnp.int32(0)

    def sel_key():
        xb = blk()
        u = lax.bitcast_convert_type(xb, jnp.int32)
        ka = u & mant
        ks = u ^ ((u >> 31) & mant)
        sel = (ka > st[_CAND]) | ((ka == st[_CAND])
                                  & (flat_idx() <= st[_ICAND]))
        return jnp.where(sel, ks, i32min), sel

    @pl.when((s >= s_b0) & (s < s_out))
    def _bounds():
        bit = jnp.int32(31) - (s - jnp.int32(s_b0))
        wk, _ = sel_key()
        for j in range(8):
            trial = st[_C8 + j] | (one << bit)
            cnt = jnp.sum((wk >= (trial ^ i32min)).astype(jnp.int32))
            new = st[_N8 + j] + cnt

            @pl.when(b != nb - 1)
            def _(j=j, new=new):
                st[_N8 + j] = new

            @pl.when(b == nb - 1)
            def _(j=j, new=new, trial=trial):
                st[_C8 + j] = jnp.where(new >= jnp.int32(targets[j]),
                                        trial, st[_C8 + j])
                st[_N8 + j] = jnp.int32(0)

    @pl.when(s == s_out)
    def _emit():
        xb = blk()
        u = lax.bitcast_convert_type(xb, jnp.int32)
        ks = u ^ ((u >> 31) & mant)
        _, sel = sel_key()

        def key_of(j):
            return st[_C8 + j] ^ i32min

        def fbits(kv):
            return kv ^ ((kv >> 31) & mant)

        mx_k = (key_of(0), key_of(2), key_of(4), key_of(6))
        mn_k = (key_of(1), key_of(3), key_of(5), key_of(7))
        j1 = ks < mn_k[0]
        j2 = ks < mn_k[1]
        j3 = ks < mn_k[2]
        mn_b = jnp.where(j1, jnp.where(j2, jnp.where(j3, fbits(mn_k[3]),
                                                     fbits(mn_k[2])),
                                       fbits(mn_k[1])), fbits(mn_k[0]))
        mx_b = jnp.where(j1, jnp.where(j2, jnp.where(j3, fbits(mx_k[3]),
                                                     fbits(mx_k[2])),
                                       fbits(mx_k[1])), fbits(mx_k[0]))
        mn_v = lax.bitcast_convert_type(mn_b, jnp.float32)
        mx_v = lax.bitcast_convert_type(mx_b, jnp.float32)
        step = (mx_v - mn_v) / jnp.float32(2 ** _BITS - 1)
        safe = jnp.where(step == 0.0, jnp.float32(1.0), step)
        q = jnp.round((xb - mn_v) / safe) * safe + mn_v
        q = jnp.where(mn_v == mx_v, xb, q)
        ov[...] = jnp.where(sel, q, jnp.float32(0.0))
        cp = pltpu.make_async_copy(ov, o_hbm.at[pl.ds(b * br, br), :],
                                   sem_out)
        cp.start()
        cp.wait()


def kernel(input):
    shape = input.shape
    n = 1
    for sz in shape:
        n *= sz
    k = int(_RATIO * n)
    assert k % _PARTITION == 0
    C = shape[-1]
    R = n // C
    br = R if R <= 512 else 512
    assert R % br == 0
    nb = R // br
    idx_bits = max(1, (n - 1).bit_length())
    n_sweeps = 31 + 1 + idx_bits + 32 + 1
    x2 = input.reshape(R, C)
    out = pl.pallas_call(
        functools.partial(_tqk_kernel, k=k, idx_bits=idx_bits, br=br, nb=nb),
        grid=(n_sweeps, nb),
        in_specs=[pl.BlockSpec(memory_space=pl.ANY)],
        out_specs=pl.BlockSpec(memory_space=pl.ANY),
        out_shape=jax.ShapeDtypeStruct((R, C), jnp.float32),
        scratch_shapes=[
            pltpu.VMEM((R, C), jnp.float32),
            pltpu.VMEM((br, C), jnp.float32),
            pltpu.SMEM((20,), jnp.int32),
            pltpu.SemaphoreType.DMA,
            pltpu.SemaphoreType.DMA,
        ],
    )(x2)
    return out.reshape(shape)
